# Initial kernel scaffold; baseline (speedup 1.0000x reference)
import jax
import jax.numpy as jnp
from jax.experimental import pallas as pl


def kernel(nodes, edges, globals_, W_node, b_node, W_edge, b_edge, W_gnode, b_gnode, W_gedge, b_gedge, W_glob, b_glob, W_final, b_final, senders, receivers, n_node, n_edge):
    raise NotImplementedError("write your pallas kernel here")


# speedup vs baseline: 1.5301x; 1.5301x over previous
import jax
import jax.numpy as jnp
from jax import lax
from jax.experimental import pallas as pl
from jax.experimental.pallas import tpu as pltpu
from jax.experimental.pallas import tpu_sc as plsc

N = 10000
E = 320000
DN = 128
DE = 16
DG = 128
DO = DN + DE

NC = 2
NS = 16
NW = NC * NS
EPW = E // NW
K = 40
NCH = EPW // K
N_PAD = 10240
RPT = N_PAD // NS

_E_BLK = 6400
_E_GRID = E // _E_BLK


def _edges_pre_body(e_ref, w_ref, b_ref, cn_ref, ce_ref, esum_ref):
    blk = e_ref[...]
    full = jnp.dot(blk, w_ref[...], preferred_element_type=jnp.float32) + b_ref[...]
    cn_ref[...] = full[:, :DN]
    ce_ref[...] = full[:, DN:]

    @pl.when(pl.program_id(0) == 0)
    def _():
        esum_ref[...] = jnp.zeros_like(esum_ref)

    esum_ref[...] += jnp.sum(blk, axis=0, keepdims=True)


_edges_pre = pl.pallas_call(
    _edges_pre_body,
    grid=(_E_GRID,),
    in_specs=[
        pl.BlockSpec((_E_BLK, DE), lambda i: (i, 0)),
        pl.BlockSpec((DE, DO), lambda i: (0, 0)),
        pl.BlockSpec((1, DO), lambda i: (0, 0)),
    ],
    out_specs=[
        pl.BlockSpec((_E_BLK, DN), lambda i: (i, 0)),
        pl.BlockSpec((_E_BLK, DE), lambda i: (i, 0)),
        pl.BlockSpec((1, DE), lambda i: (0, 0)),
    ],
    out_shape=[
        jax.ShapeDtypeStruct((E, DN), jnp.float32),
        jax.ShapeDtypeStruct((E, DE), jnp.float32),
        jax.ShapeDtypeStruct((1, DE), jnp.float32),
    ],
)


def _leaky(x):
    return jnp.where(x >= 0, x, 0.01 * x)


def _tables_body(nodes_ref, ws_ref, wr_ref, esum_ref, glob_ref,
                 wgn_ref, bgn_ref, wge_ref, bge_ref, wgg_ref, bgg_ref,
                 wf_ref, bf_ref, ts_ref, tr_ref, gout_ref):
    nd = nodes_ref[...]
    ts_ref[...] = jnp.dot(nd, ws_ref[...], preferred_element_type=jnp.float32)
    tr_ref[...] = jnp.dot(nd, wr_ref[...], preferred_element_type=jnp.float32)
    nsum = jnp.sum(nd, axis=0, keepdims=True)
    tmp_node = _leaky(
        jnp.dot(nsum, wgn_ref[...], preferred_element_type=jnp.float32) + bgn_ref[...])
    tmp_edge = _leaky(
        jnp.dot(esum_ref[...], wge_ref[...], preferred_element_type=jnp.float32)
        + bge_ref[...])
    tmp_glob = _leaky(
        jnp.dot(glob_ref[...], wgg_ref[...], preferred_element_type=jnp.float32)
        + bgg_ref[...])
    fargs = jnp.concatenate([tmp_glob, tmp_node, tmp_edge], axis=1)
    gout_ref[...] = _leaky(
        jnp.dot(fargs, wf_ref[...], preferred_element_type=jnp.float32) + bf_ref[...])


_tables = pl.pallas_call(
    _tables_body,
    out_shape=[
        jax.ShapeDtypeStruct((N, DO), jnp.float32),
        jax.ShapeDtypeStruct((N, DO), jnp.float32),
        jax.ShapeDtypeStruct((1, DG), jnp.float32),
    ],
)


def _sc_body(ts_hbm, tr_hbm, cn_hbm, ce_hbm, snd_hbm, rcv_hbm,
             eout_hbm, part_hbm,
             idx_s_all, idx_r_all, s_buf, r_buf, m_buf, e_buf, accum,
             sem_s, sem_r):
    cid = lax.axis_index("c")
    sid = lax.axis_index("s")
    wid = cid * NS + sid
    row0 = sid * RPT

    def _zrow(i, _):
        for g in range(DN // 16):
            m_buf[i, pl.ds(g * 16, 16)] = jnp.zeros((16,), jnp.float32)
        return 0

    lax.fori_loop(0, K, _zrow, 0)
    for j in range(RPT // K):
        pltpu.sync_copy(m_buf, accum.at[pl.ds(row0 + j * K, K)])
    rem = RPT % K
    if rem:
        pltpu.sync_copy(m_buf.at[pl.ds(0, rem)],
                        accum.at[pl.ds(row0 + (RPT // K) * K, rem)])

    pltpu.sync_copy(snd_hbm.at[wid], idx_s_all)
    pltpu.sync_copy(rcv_hbm.at[wid], idx_r_all)
    plsc.subcore_barrier()

    ebase = wid * EPW

    def _chunk(i, _):
        idx_s = idx_s_all.at[i]
        idx_r = idx_r_all.at[i]
        cp_s = pltpu.async_copy(ts_hbm.at[idx_s], s_buf, sem_s)
        cp_r = pltpu.async_copy(tr_hbm.at[idx_r], r_buf, sem_r)
        pltpu.sync_copy(cn_hbm.at[pl.ds(ebase + i * K, K)], m_buf)
        pltpu.sync_copy(ce_hbm.at[pl.ds(ebase + i * K, K)], e_buf)
        cp_s.wait()
        cp_r.wait()

        def _edge(e, _):
            for g in range(DN // 16):
                sl = pl.ds(g * 16, 16)
                x = m_buf[e, sl] + s_buf[e, sl] + r_buf[e, sl]
                m_buf[e, sl] = jnp.maximum(x, 0.01 * x)
            sl16 = pl.ds(0, 16)
            sl_hi = pl.ds(DN, 16)
            x = e_buf[e, sl16] + s_buf[e, sl_hi] + r_buf[e, sl_hi]
            e_buf[e, sl16] = jnp.maximum(x, 0.01 * x)
            return 0

        lax.fori_loop(0, K, _edge, 0)
        pltpu.sync_copy(m_buf, accum.at[idx_r], add=True)
        pltpu.sync_copy(e_buf, eout_hbm.at[pl.ds(ebase + i * K, K)])
        return 0

    lax.fori_loop(0, NCH, _chunk, 0)
    plsc.subcore_barrier()
    pltpu.sync_copy(accum.at[pl.ds(row0, RPT)],
                    part_hbm.at[pl.ds(cid * N_PAD + row0, RPT)])


_sc_gather_scatter = pl.kernel(
    _sc_body,
    out_type=[
        jax.ShapeDtypeStruct((E, DE), jnp.float32),
        jax.ShapeDtypeStruct((2 * N_PAD, DN), jnp.float32),
    ],
    mesh=plsc.VectorSubcoreMesh(core_axis_name="c", subcore_axis_name="s"),
    compiler_params=pltpu.CompilerParams(use_tc_tiling_on_sc=False),
    scratch_types=[
        pltpu.VMEM((NCH, K), jnp.int32),
        pltpu.VMEM((NCH, K), jnp.int32),
        pltpu.VMEM((K, DO), jnp.float32),
        pltpu.VMEM((K, DO), jnp.float32),
        pltpu.VMEM((K, DN), jnp.float32),
        pltpu.VMEM((K, DE), jnp.float32),
        pltpu.VMEM_SHARED((N_PAD, DN), jnp.float32),
        pltpu.SemaphoreType.DMA,
        pltpu.SemaphoreType.DMA,
    ],
)


_N_BLK = 80


def _combine_body(a_ref, b_ref, o_ref):
    o_ref[...] = a_ref[...] + b_ref[...]


_combine = pl.pallas_call(
    _combine_body,
    grid=(N // _N_BLK,),
    in_specs=[
        pl.BlockSpec((_N_BLK, DN), lambda i: (i, 0)),
        pl.BlockSpec((_N_BLK, DN), lambda i: (N_PAD // _N_BLK + i, 0)),
    ],
    out_specs=pl.BlockSpec((_N_BLK, DN), lambda i: (i, 0)),
    out_shape=jax.ShapeDtypeStruct((N, DN), jnp.float32),
)


def kernel(nodes, edges, globals_, W_node, b_node, W_edge, b_edge,
           W_gnode, b_gnode, W_gedge, b_gedge, W_glob, b_glob,
           W_final, b_final, senders, receivers, n_node, n_edge):
    W_s = jnp.concatenate([W_node[:DN], W_edge[:DN]], axis=1)
    W_r = jnp.concatenate([W_node[DN:2 * DN], W_edge[DN:2 * DN]], axis=1)
    W_e = jnp.concatenate([W_node[2 * DN:], W_edge[2 * DN:]], axis=1)
    b_all = jnp.concatenate([b_node, b_edge]).reshape(1, DO)

    CN, CE, esum = _edges_pre(edges, W_e, b_all)
    TS, TR, new_global = _tables(
        nodes, W_s, W_r, esum, globals_,
        W_gnode, b_gnode.reshape(1, DG), W_gedge, b_gedge.reshape(1, DG),
        W_glob, b_glob.reshape(1, DG), W_final, b_final.reshape(1, DG))

    snd = senders.astype(jnp.int32).reshape(NW, NCH, K)
    rcv = receivers.astype(jnp.int32).reshape(NW, NCH, K)
    new_edges, partials = _sc_gather_scatter(TS, TR, CN, CE, snd, rcv)
    new_nodes = _combine(partials, partials)
    return new_nodes, new_edges, new_global
